# Initial kernel scaffold; baseline (speedup 1.0000x reference)
import jax
import jax.numpy as jnp
from jax.experimental import pallas as pl


def kernel(feats, adj_edge_index_0, adj_edge_index_1, edge_index_0, edge_index_1, W0, W1, W_edge, b_edge):
    raise NotImplementedError("write your pallas kernel here")


# speedup vs baseline: 6.3505x; 6.3505x over previous
import functools

import jax
import jax.numpy as jnp
from jax import lax
from jax.experimental import pallas as pl
from jax.experimental.pallas import tpu as pltpu
from jax.experimental.pallas import tpu_sc as plsc

N = 10000
E = 320000
D = 128

NS = 16
LANES = 16

N_PAD = 10240
PAD_ROW = N
E_PAD = 327680

BLK = 128
NBLK = E_PAD // BLK
BLK_PER_TILE = NBLK // NS
IDX_GRP = 32
N_GRP = BLK_PER_TILE // IDX_GRP

ROWS_PER_TILE = N_PAD // NS
ROW_CHUNK = 128
N_ROW_CHUNKS = ROWS_PER_TILE // ROW_CHUNK

EPT = E_PAD // NS
ECH = 2048
N_ECH = EPT // ECH
IT_LOGITS = ECH // LANES


def _tc_encode_matmul(feats, W0, W1):
    def body(f_ref, w0_ref, w1_ref, h0_ref, h1_ref):
        f = f_ref[...]
        ztail = jnp.zeros((N_PAD - N, D), jnp.float32)
        h0_ref[pl.ds(0, N), :] = jnp.dot(f, w0_ref[...],
                                         preferred_element_type=jnp.float32)
        h1_ref[pl.ds(0, N), :] = jnp.dot(f, w1_ref[...],
                                         preferred_element_type=jnp.float32)
        h0_ref[pl.ds(N, N_PAD - N), :] = ztail
        h1_ref[pl.ds(N, N_PAD - N), :] = ztail

    return pl.pallas_call(
        body,
        out_shape=[jax.ShapeDtypeStruct((N_PAD, D), jnp.float32)] * 2,
    )(feats, W0, W1)


def _tc_relu_proj(agg0, agg1, We2T, b2):
    dn = (((1,), (1,)), ((), ()))

    def body(a0_ref, a1_ref, w_ref, b_ref, s0_ref, s1_ref):
        w = w_ref[...]
        bias = jnp.where(
            lax.broadcasted_iota(jnp.int32, (2, N_PAD), 0) == 0, b_ref[0, 0], 0.0
        )
        r0 = jnp.maximum(a0_ref[...], 0.0)
        r1 = jnp.maximum(a1_ref[...], 0.0)
        s0_ref[...] = lax.dot_general(
            w, r0, dn, preferred_element_type=jnp.float32) + bias
        s1_ref[...] = lax.dot_general(
            w, r1, dn, preferred_element_type=jnp.float32) + bias

    return pl.pallas_call(
        body,
        out_shape=[jax.ShapeDtypeStruct((2, N_PAD), jnp.float32)] * 2,
    )(agg0, agg1, We2T, b2)


def _sc_segment_sum(h0, h1, src0, dst0, src1, dst1):
    mesh = plsc.VectorSubcoreMesh(core_axis_name="c", subcore_axis_name="s")

    @functools.partial(
        pl.kernel,
        out_type=[jax.ShapeDtypeStruct((N_PAD, D), jnp.float32)] * 2,
        mesh=mesh,
        scratch_types=[
            pltpu.VMEM_SHARED((N_PAD, D), jnp.float32),
            pltpu.VMEM((IDX_GRP, BLK), jnp.int32),
            pltpu.VMEM((IDX_GRP, BLK), jnp.int32),
            pltpu.VMEM((BLK, D), jnp.float32),
            pltpu.SemaphoreType.DMA,
        ],
    )
    def k(h0_hbm, h1_hbm, s0_hbm, d0_hbm, s1_hbm, d1_hbm,
          agg0_hbm, agg1_hbm,
          acc, sidx, didx, rows, sem):
        c = lax.axis_index("c")
        s = lax.axis_index("s")

        def zfill(i, carry):
            r = i // (D // LANES)
            col = (i % (D // LANES)) * LANES
            rows[r, pl.ds(col, LANES)] = jnp.zeros((LANES,), jnp.float32)
            return carry

        lax.fori_loop(0, ROW_CHUNK * (D // LANES), zfill, 0)
        row0 = s * ROWS_PER_TILE

        def zcopy(j, carry):
            pltpu.sync_copy(rows, acc.at[pl.ds(row0 + j * ROW_CHUNK, ROW_CHUNK)])
            return carry

        lax.fori_loop(0, N_ROW_CHUNKS, zcopy, 0)
        plsc.subcore_barrier()

        blk0 = s * BLK_PER_TILE

        def edge_loop(h_hbm, src_hbm, dst_hbm):
            def grp_body(g, carry):
                gb = blk0 + g * IDX_GRP
                pltpu.sync_copy(src_hbm.at[pl.ds(gb, IDX_GRP)], sidx)
                pltpu.sync_copy(dst_hbm.at[pl.ds(gb, IDX_GRP)], didx)

                def blk_body(i, carry2):
                    pltpu.async_copy(h_hbm.at[sidx.at[i]], rows, sem).wait()
                    pltpu.sync_copy(rows, acc.at[didx.at[i]], add=True)
                    return carry2

                lax.fori_loop(0, IDX_GRP, blk_body, 0)
                return carry

            lax.fori_loop(0, N_GRP, grp_body, 0)

        @pl.when(c == 0)
        def _():
            edge_loop(h0_hbm, s0_hbm, d0_hbm)

        @pl.when(c == 1)
        def _():
            edge_loop(h1_hbm, s1_hbm, d1_hbm)

        plsc.subcore_barrier()

        def copy_out(agg_hbm):
            def cp(j, carry):
                r = row0 + j * ROW_CHUNK
                pltpu.sync_copy(acc.at[pl.ds(r, ROW_CHUNK)],
                                agg_hbm.at[pl.ds(r, ROW_CHUNK)])
                return carry

            lax.fori_loop(0, N_ROW_CHUNKS, cp, 0)

        @pl.when(c == 0)
        def _():
            copy_out(agg0_hbm)

        @pl.when(c == 1)
        def _():
            copy_out(agg1_hbm)

    return k(h0, h1, src0, dst0, src1, dst1)


def _sc_edge_logits(s0, s1, es0, ed0, es1, ed1):
    mesh = plsc.VectorSubcoreMesh(core_axis_name="c", subcore_axis_name="s")

    @functools.partial(
        pl.kernel,
        out_type=[jax.ShapeDtypeStruct((E_PAD,), jnp.float32)] * 2,
        mesh=mesh,
        compiler_params=pltpu.CompilerParams(needs_layout_passes=False),
        scratch_types=[
            pltpu.VMEM((2 * N_PAD,), jnp.float32),
            pltpu.VMEM((ECH,), jnp.int32),
            pltpu.VMEM((ECH,), jnp.int32),
            pltpu.VMEM((ECH,), jnp.float32),
        ],
    )
    def k(s0_hbm, s1_hbm, es0_hbm, ed0_hbm, es1_hbm, ed1_hbm,
          out0_hbm, out1_hbm,
          stab, sbuf, dbuf, obuf):
        c = lax.axis_index("c")
        s = lax.axis_index("s")
        base = s * EPT

        def view_loop(s_hbm, es_hbm, ed_hbm, out_hbm):
            pltpu.sync_copy(s_hbm, stab)

            def chunk(cc, carry):
                cb = base + cc * ECH
                pltpu.sync_copy(es_hbm.at[pl.ds(cb, ECH)], sbuf)
                pltpu.sync_copy(ed_hbm.at[pl.ds(cb, ECH)], dbuf)

                def it(i, carry2):
                    off = pl.multiple_of(i * LANES, LANES)
                    si = sbuf[pl.ds(off, LANES)]
                    di = dbuf[pl.ds(off, LANES)] + N_PAD
                    g1 = plsc.load_gather(stab, [si])
                    g2 = plsc.load_gather(stab, [di])
                    obuf[pl.ds(off, LANES)] = g1 + g2
                    return carry2

                lax.fori_loop(0, IT_LOGITS, it, 0)
                pltpu.sync_copy(obuf, out_hbm.at[pl.ds(cb, ECH)])
                return carry

            lax.fori_loop(0, N_ECH, chunk, 0)

        @pl.when(c == 0)
        def _():
            view_loop(s0_hbm, es0_hbm, ed0_hbm, out0_hbm)

        @pl.when(c == 1)
        def _():
            view_loop(s1_hbm, es1_hbm, ed1_hbm, out1_hbm)

    return k(s0, s1, es0, ed0, es1, ed1)


def _pad_idx(x):
    return jnp.concatenate(
        [x, jnp.full((E_PAD - E,), PAD_ROW, dtype=jnp.int32)]
    )


def kernel(feats, adj_edge_index_0, adj_edge_index_1, edge_index_0,
           edge_index_1, W0, W1, W_edge, b_edge):
    adj0 = adj_edge_index_0.astype(jnp.int32)
    adj1 = adj_edge_index_1.astype(jnp.int32)
    ei0 = edge_index_0.astype(jnp.int32)
    ei1 = edge_index_1.astype(jnp.int32)

    h0, h1 = _tc_encode_matmul(feats, W0, W1)

    src0 = _pad_idx(adj0[0]).reshape(NBLK, BLK)
    dst0 = _pad_idx(adj0[1]).reshape(NBLK, BLK)
    src1 = _pad_idx(adj1[0]).reshape(NBLK, BLK)
    dst1 = _pad_idx(adj1[1]).reshape(NBLK, BLK)
    agg0, agg1 = _sc_segment_sum(h0, h1, src0, dst0, src1, dst1)

    We2T = W_edge.reshape(2, D)
    b2 = b_edge.reshape(1, 1)
    s0, s1 = _tc_relu_proj(agg0, agg1, We2T, b2)

    l0, l1 = _sc_edge_logits(s0.reshape(2 * N_PAD), s1.reshape(2 * N_PAD),
                             _pad_idx(ei0[0]), _pad_idx(ei0[1]),
                             _pad_idx(ei1[0]), _pad_idx(ei1[1]))
    return (l0[:E].reshape(E, 1), l1[:E].reshape(E, 1))


# speedup vs baseline: 7.4831x; 1.1783x over previous
import functools

import jax
import jax.numpy as jnp
from jax import lax
from jax.experimental import pallas as pl
from jax.experimental.pallas import tpu as pltpu
from jax.experimental.pallas import tpu_sc as plsc

N = 10000
E = 320000
D = 128

NS = 16
LANES = 16

N_PAD = 10240
PAD_ROW = N
E_PAD = 327680

BLK = 128
NBLK = E_PAD // BLK
BLK_PER_TILE = NBLK // NS
IDX_GRP = 32
N_GRP = BLK_PER_TILE // IDX_GRP

ROWS_PER_TILE = N_PAD // NS
ROW_CHUNK = 128
N_ROW_CHUNKS = ROWS_PER_TILE // ROW_CHUNK

EPT = E_PAD // NS
ECH = 2048
N_ECH = EPT // ECH
IT_LOGITS = ECH // LANES


def _tc_encode_matmul(feats, W0, W1):
    def body(f_ref, w0_ref, w1_ref, h0_ref, h1_ref):
        f = f_ref[...]
        ztail = jnp.zeros((N_PAD - N, D), jnp.float32)
        h0_ref[pl.ds(0, N), :] = jnp.dot(f, w0_ref[...],
                                         preferred_element_type=jnp.float32)
        h1_ref[pl.ds(0, N), :] = jnp.dot(f, w1_ref[...],
                                         preferred_element_type=jnp.float32)
        h0_ref[pl.ds(N, N_PAD - N), :] = ztail
        h1_ref[pl.ds(N, N_PAD - N), :] = ztail

    return pl.pallas_call(
        body,
        out_shape=[jax.ShapeDtypeStruct((N_PAD, D), jnp.float32)] * 2,
    )(feats, W0, W1)


def _tc_relu_proj(agg0, agg1, We2T, b2):
    dn = (((1,), (1,)), ((), ()))

    def body(a0_ref, a1_ref, w_ref, b_ref, s0_ref, s1_ref):
        w = w_ref[...]
        bias = jnp.where(
            lax.broadcasted_iota(jnp.int32, (2, N_PAD), 0) == 0, b_ref[0, 0], 0.0
        )
        r0 = jnp.maximum(a0_ref[...], 0.0)
        r1 = jnp.maximum(a1_ref[...], 0.0)
        s0_ref[...] = lax.dot_general(
            w, r0, dn, preferred_element_type=jnp.float32) + bias
        s1_ref[...] = lax.dot_general(
            w, r1, dn, preferred_element_type=jnp.float32) + bias

    return pl.pallas_call(
        body,
        out_shape=[jax.ShapeDtypeStruct((2, N_PAD), jnp.float32)] * 2,
    )(agg0, agg1, We2T, b2)


def _sc_segment_sum(h0, h1, src0, dst0, src1, dst1):
    mesh = plsc.VectorSubcoreMesh(core_axis_name="c", subcore_axis_name="s")

    @functools.partial(
        pl.kernel,
        out_type=[jax.ShapeDtypeStruct((N_PAD, D), jnp.float32)] * 2,
        mesh=mesh,
        scratch_types=[
            pltpu.VMEM_SHARED((N_PAD, D), jnp.float32),
            pltpu.VMEM((IDX_GRP, BLK), jnp.int32),
            pltpu.VMEM((IDX_GRP, BLK), jnp.int32),
            pltpu.VMEM((BLK, D), jnp.float32),
            pltpu.VMEM((BLK, D), jnp.float32),
            pltpu.SemaphoreType.DMA,
            pltpu.SemaphoreType.DMA,
        ],
    )
    def k(h0_hbm, h1_hbm, s0_hbm, d0_hbm, s1_hbm, d1_hbm,
          agg0_hbm, agg1_hbm,
          acc, sidx, didx, rows, rows1, sem, sem1):
        c = lax.axis_index("c")
        s = lax.axis_index("s")

        def zfill(i, carry):
            r = i // (D // LANES)
            col = (i % (D // LANES)) * LANES
            rows[r, pl.ds(col, LANES)] = jnp.zeros((LANES,), jnp.float32)
            return carry

        lax.fori_loop(0, ROW_CHUNK * (D // LANES), zfill, 0)
        row0 = s * ROWS_PER_TILE

        def zcopy(j, carry):
            pltpu.sync_copy(rows, acc.at[pl.ds(row0 + j * ROW_CHUNK, ROW_CHUNK)])
            return carry

        lax.fori_loop(0, N_ROW_CHUNKS, zcopy, 0)
        plsc.subcore_barrier()

        blk0 = s * BLK_PER_TILE

        def edge_loop(h_hbm, src_hbm, dst_hbm):
            def gather(i, buf, sm):
                return pltpu.make_async_copy(h_hbm.at[sidx.at[i]], buf, sm)

            def grp_body(g, carry):
                gb = blk0 + g * IDX_GRP
                pltpu.sync_copy(src_hbm.at[pl.ds(gb, IDX_GRP)], sidx)
                pltpu.sync_copy(dst_hbm.at[pl.ds(gb, IDX_GRP)], didx)
                gather(0, rows, sem).start()
                gather(1, rows1, sem1).start()

                def pair(j, carry2):
                    b0 = 2 * j
                    gather(b0, rows, sem).wait()
                    pltpu.sync_copy(rows, acc.at[didx.at[b0]], add=True)

                    @pl.when(b0 + 2 < IDX_GRP)
                    def _():
                        gather(b0 + 2, rows, sem).start()

                    gather(b0 + 1, rows1, sem1).wait()
                    pltpu.sync_copy(rows1, acc.at[didx.at[b0 + 1]], add=True)

                    @pl.when(b0 + 3 < IDX_GRP)
                    def _():
                        gather(b0 + 3, rows1, sem1).start()

                    return carry2

                lax.fori_loop(0, IDX_GRP // 2, pair, 0)
                return carry

            lax.fori_loop(0, N_GRP, grp_body, 0)

        @pl.when(c == 0)
        def _():
            edge_loop(h0_hbm, s0_hbm, d0_hbm)

        @pl.when(c == 1)
        def _():
            edge_loop(h1_hbm, s1_hbm, d1_hbm)

        plsc.subcore_barrier()

        def copy_out(agg_hbm):
            def cp(j, carry):
                r = row0 + j * ROW_CHUNK
                pltpu.sync_copy(acc.at[pl.ds(r, ROW_CHUNK)],
                                agg_hbm.at[pl.ds(r, ROW_CHUNK)])
                return carry

            lax.fori_loop(0, N_ROW_CHUNKS, cp, 0)

        @pl.when(c == 0)
        def _():
            copy_out(agg0_hbm)

        @pl.when(c == 1)
        def _():
            copy_out(agg1_hbm)

    return k(h0, h1, src0, dst0, src1, dst1)


def _sc_edge_logits(s0, s1, es0, ed0, es1, ed1):
    mesh = plsc.VectorSubcoreMesh(core_axis_name="c", subcore_axis_name="s")

    @functools.partial(
        pl.kernel,
        out_type=[jax.ShapeDtypeStruct((E_PAD,), jnp.float32)] * 2,
        mesh=mesh,
        compiler_params=pltpu.CompilerParams(needs_layout_passes=False),
        scratch_types=[
            pltpu.VMEM((2 * N_PAD,), jnp.float32),
            pltpu.VMEM((ECH,), jnp.int32),
            pltpu.VMEM((ECH,), jnp.int32),
            pltpu.VMEM((ECH,), jnp.float32),
        ],
    )
    def k(s0_hbm, s1_hbm, es0_hbm, ed0_hbm, es1_hbm, ed1_hbm,
          out0_hbm, out1_hbm,
          stab, sbuf, dbuf, obuf):
        c = lax.axis_index("c")
        s = lax.axis_index("s")
        base = s * EPT

        def view_loop(s_hbm, es_hbm, ed_hbm, out_hbm):
            pltpu.sync_copy(s_hbm, stab)

            def chunk(cc, carry):
                cb = base + cc * ECH
                pltpu.sync_copy(es_hbm.at[pl.ds(cb, ECH)], sbuf)
                pltpu.sync_copy(ed_hbm.at[pl.ds(cb, ECH)], dbuf)

                def it(i, carry2):
                    off = pl.multiple_of(i * LANES, LANES)
                    si = sbuf[pl.ds(off, LANES)]
                    di = dbuf[pl.ds(off, LANES)] + N_PAD
                    g1 = plsc.load_gather(stab, [si])
                    g2 = plsc.load_gather(stab, [di])
                    obuf[pl.ds(off, LANES)] = g1 + g2
                    return carry2

                lax.fori_loop(0, IT_LOGITS, it, 0)
                pltpu.sync_copy(obuf, out_hbm.at[pl.ds(cb, ECH)])
                return carry

            lax.fori_loop(0, N_ECH, chunk, 0)

        @pl.when(c == 0)
        def _():
            view_loop(s0_hbm, es0_hbm, ed0_hbm, out0_hbm)

        @pl.when(c == 1)
        def _():
            view_loop(s1_hbm, es1_hbm, ed1_hbm, out1_hbm)

    return k(s0, s1, es0, ed0, es1, ed1)


def _pad_idx(x):
    return jnp.concatenate(
        [x, jnp.full((E_PAD - E,), PAD_ROW, dtype=jnp.int32)]
    )


def kernel(feats, adj_edge_index_0, adj_edge_index_1, edge_index_0,
           edge_index_1, W0, W1, W_edge, b_edge):
    adj0 = adj_edge_index_0.astype(jnp.int32)
    adj1 = adj_edge_index_1.astype(jnp.int32)
    ei0 = edge_index_0.astype(jnp.int32)
    ei1 = edge_index_1.astype(jnp.int32)

    h0, h1 = _tc_encode_matmul(feats, W0, W1)

    src0 = _pad_idx(adj0[0]).reshape(NBLK, BLK)
    dst0 = _pad_idx(adj0[1]).reshape(NBLK, BLK)
    src1 = _pad_idx(adj1[0]).reshape(NBLK, BLK)
    dst1 = _pad_idx(adj1[1]).reshape(NBLK, BLK)
    agg0, agg1 = _sc_segment_sum(h0, h1, src0, dst0, src1, dst1)

    We2T = W_edge.reshape(2, D)
    b2 = b_edge.reshape(1, 1)
    s0, s1 = _tc_relu_proj(agg0, agg1, We2T, b2)

    l0, l1 = _sc_edge_logits(s0.reshape(2 * N_PAD), s1.reshape(2 * N_PAD),
                             _pad_idx(ei0[0]), _pad_idx(ei0[1]),
                             _pad_idx(ei1[0]), _pad_idx(ei1[1]))
    return (l0[:E].reshape(E, 1), l1[:E].reshape(E, 1))


# speedup vs baseline: 8.4111x; 1.1240x over previous
import functools

import jax
import jax.numpy as jnp
from jax import lax
from jax.experimental import pallas as pl
from jax.experimental.pallas import tpu as pltpu
from jax.experimental.pallas import tpu_sc as plsc

N = 10000
E = 320000
D = 128

NS = 16
LANES = 16

N_PAD = 10240
PAD_ROW = N
E_PAD = 327680

BLK = 64
NBLK = E_PAD // BLK
BLK_PER_TILE = NBLK // NS
IDX_GRP = 64
N_GRP = BLK_PER_TILE // IDX_GRP
NBUF = 4

ROWS_PER_TILE = N_PAD // NS
ROW_CHUNK = 64
N_ROW_CHUNKS = ROWS_PER_TILE // ROW_CHUNK

EPT = E_PAD // NS
ECH = 2048
N_ECH = EPT // ECH
IT_LOGITS = ECH // LANES


def _tc_encode_matmul(feats, W0, W1):
    def body(f_ref, w0_ref, w1_ref, h0_ref, h1_ref):
        f = f_ref[...]
        ztail = jnp.zeros((N_PAD - N, D), jnp.float32)
        h0_ref[pl.ds(0, N), :] = jnp.dot(f, w0_ref[...],
                                         preferred_element_type=jnp.float32)
        h1_ref[pl.ds(0, N), :] = jnp.dot(f, w1_ref[...],
                                         preferred_element_type=jnp.float32)
        h0_ref[pl.ds(N, N_PAD - N), :] = ztail
        h1_ref[pl.ds(N, N_PAD - N), :] = ztail

    return pl.pallas_call(
        body,
        out_shape=[jax.ShapeDtypeStruct((N_PAD, D), jnp.float32)] * 2,
    )(feats, W0, W1)


def _tc_relu_proj(agg0, agg1, We2T, b2):
    dn = (((1,), (1,)), ((), ()))

    def body(a0_ref, a1_ref, w_ref, b_ref, s0_ref, s1_ref):
        w = w_ref[...]
        bias = jnp.where(
            lax.broadcasted_iota(jnp.int32, (2, N_PAD), 0) == 0, b_ref[0, 0], 0.0
        )
        r0 = jnp.maximum(a0_ref[...], 0.0)
        r1 = jnp.maximum(a1_ref[...], 0.0)
        s0_ref[...] = lax.dot_general(
            w, r0, dn, preferred_element_type=jnp.float32) + bias
        s1_ref[...] = lax.dot_general(
            w, r1, dn, preferred_element_type=jnp.float32) + bias

    return pl.pallas_call(
        body,
        out_shape=[jax.ShapeDtypeStruct((2, N_PAD), jnp.float32)] * 2,
    )(agg0, agg1, We2T, b2)


def _sc_segment_sum(h0, h1, src0, dst0, src1, dst1):
    mesh = plsc.VectorSubcoreMesh(core_axis_name="c", subcore_axis_name="s")

    @functools.partial(
        pl.kernel,
        out_type=[jax.ShapeDtypeStruct((N_PAD, D), jnp.float32)] * 2,
        mesh=mesh,
        scratch_types=[
            pltpu.VMEM_SHARED((N_PAD, D), jnp.float32),
            pltpu.VMEM((IDX_GRP, BLK), jnp.int32),
            pltpu.VMEM((IDX_GRP, BLK), jnp.int32),
            [pltpu.VMEM((BLK, D), jnp.float32)] * NBUF,
            [pltpu.SemaphoreType.DMA] * NBUF,
        ],
    )
    def k(h0_hbm, h1_hbm, s0_hbm, d0_hbm, s1_hbm, d1_hbm,
          agg0_hbm, agg1_hbm,
          acc, sidx, didx, rings, sems):
        c = lax.axis_index("c")
        s = lax.axis_index("s")

        def zfill(i, carry):
            r = i // (D // LANES)
            col = (i % (D // LANES)) * LANES
            rings[0][r, pl.ds(col, LANES)] = jnp.zeros((LANES,), jnp.float32)
            return carry

        lax.fori_loop(0, ROW_CHUNK * (D // LANES), zfill, 0)
        row0 = s * ROWS_PER_TILE

        def zcopy(j, carry):
            pltpu.sync_copy(rings[0],
                            acc.at[pl.ds(row0 + j * ROW_CHUNK, ROW_CHUNK)])
            return carry

        lax.fori_loop(0, N_ROW_CHUNKS, zcopy, 0)
        plsc.subcore_barrier()

        blk0 = s * BLK_PER_TILE

        def edge_loop(h_hbm, src_hbm, dst_hbm):
            def gather(i, b):
                return pltpu.make_async_copy(h_hbm.at[sidx.at[i]],
                                             rings[b], sems[b])

            def grp_body(g, carry):
                gb = blk0 + g * IDX_GRP
                pltpu.sync_copy(src_hbm.at[pl.ds(gb, IDX_GRP)], sidx)
                pltpu.sync_copy(dst_hbm.at[pl.ds(gb, IDX_GRP)], didx)
                for b in range(NBUF - 1):
                    gather(b, b).start()

                def step(j, carry2):
                    i0 = j * NBUF
                    for b in range(NBUF):
                        i = i0 + b
                        gather(i, b).wait()
                        pltpu.sync_copy(rings[b], acc.at[didx.at[i]], add=True)

                        @pl.when(i + NBUF - 1 < IDX_GRP)
                        def _():
                            gather(i + NBUF - 1, (b + NBUF - 1) % NBUF).start()

                    return carry2

                lax.fori_loop(0, IDX_GRP // NBUF, step, 0)
                return carry

            lax.fori_loop(0, N_GRP, grp_body, 0)

        @pl.when(c == 0)
        def _():
            edge_loop(h0_hbm, s0_hbm, d0_hbm)

        @pl.when(c == 1)
        def _():
            edge_loop(h1_hbm, s1_hbm, d1_hbm)

        plsc.subcore_barrier()

        def copy_out(agg_hbm):
            def cp(j, carry):
                r = row0 + j * ROW_CHUNK
                pltpu.sync_copy(acc.at[pl.ds(r, ROW_CHUNK)],
                                agg_hbm.at[pl.ds(r, ROW_CHUNK)])
                return carry

            lax.fori_loop(0, N_ROW_CHUNKS, cp, 0)

        @pl.when(c == 0)
        def _():
            copy_out(agg0_hbm)

        @pl.when(c == 1)
        def _():
            copy_out(agg1_hbm)

    return k(h0, h1, src0, dst0, src1, dst1)


def _sc_edge_logits(s0, s1, es0, ed0, es1, ed1):
    mesh = plsc.VectorSubcoreMesh(core_axis_name="c", subcore_axis_name="s")

    @functools.partial(
        pl.kernel,
        out_type=[jax.ShapeDtypeStruct((E_PAD,), jnp.float32)] * 2,
        mesh=mesh,
        compiler_params=pltpu.CompilerParams(needs_layout_passes=False),
        scratch_types=[
            pltpu.VMEM((2 * N_PAD,), jnp.float32),
            pltpu.VMEM((ECH,), jnp.int32),
            pltpu.VMEM((ECH,), jnp.int32),
            pltpu.VMEM((ECH,), jnp.float32),
        ],
    )
    def k(s0_hbm, s1_hbm, es0_hbm, ed0_hbm, es1_hbm, ed1_hbm,
          out0_hbm, out1_hbm,
          stab, sbuf, dbuf, obuf):
        c = lax.axis_index("c")
        s = lax.axis_index("s")
        base = s * EPT

        def view_loop(s_hbm, es_hbm, ed_hbm, out_hbm):
            pltpu.sync_copy(s_hbm, stab)

            def chunk(cc, carry):
                cb = base + cc * ECH
                pltpu.sync_copy(es_hbm.at[pl.ds(cb, ECH)], sbuf)
                pltpu.sync_copy(ed_hbm.at[pl.ds(cb, ECH)], dbuf)

                def it(i, carry2):
                    off = pl.multiple_of(i * LANES, LANES)
                    si = sbuf[pl.ds(off, LANES)]
                    di = dbuf[pl.ds(off, LANES)] + N_PAD
                    g1 = plsc.load_gather(stab, [si])
                    g2 = plsc.load_gather(stab, [di])
                    obuf[pl.ds(off, LANES)] = g1 + g2
                    return carry2

                lax.fori_loop(0, IT_LOGITS, it, 0)
                pltpu.sync_copy(obuf, out_hbm.at[pl.ds(cb, ECH)])
                return carry

            lax.fori_loop(0, N_ECH, chunk, 0)

        @pl.when(c == 0)
        def _():
            view_loop(s0_hbm, es0_hbm, ed0_hbm, out0_hbm)

        @pl.when(c == 1)
        def _():
            view_loop(s1_hbm, es1_hbm, ed1_hbm, out1_hbm)

    return k(s0, s1, es0, ed0, es1, ed1)


def _pad_idx(x):
    return jnp.concatenate(
        [x, jnp.full((E_PAD - E,), PAD_ROW, dtype=jnp.int32)]
    )


def kernel(feats, adj_edge_index_0, adj_edge_index_1, edge_index_0,
           edge_index_1, W0, W1, W_edge, b_edge):
    adj0 = adj_edge_index_0.astype(jnp.int32)
    adj1 = adj_edge_index_1.astype(jnp.int32)
    ei0 = edge_index_0.astype(jnp.int32)
    ei1 = edge_index_1.astype(jnp.int32)

    h0, h1 = _tc_encode_matmul(feats, W0, W1)

    src0 = _pad_idx(adj0[0]).reshape(NBLK, BLK)
    dst0 = _pad_idx(adj0[1]).reshape(NBLK, BLK)
    src1 = _pad_idx(adj1[0]).reshape(NBLK, BLK)
    dst1 = _pad_idx(adj1[1]).reshape(NBLK, BLK)
    agg0, agg1 = _sc_segment_sum(h0, h1, src0, dst0, src1, dst1)

    We2T = W_edge.reshape(2, D)
    b2 = b_edge.reshape(1, 1)
    s0, s1 = _tc_relu_proj(agg0, agg1, We2T, b2)

    l0, l1 = _sc_edge_logits(s0.reshape(2 * N_PAD), s1.reshape(2 * N_PAD),
                             _pad_idx(ei0[0]), _pad_idx(ei0[1]),
                             _pad_idx(ei1[0]), _pad_idx(ei1[1]))
    return (l0[:E].reshape(E, 1), l1[:E].reshape(E, 1))
